# baseline (speedup 1.0000x reference)
import functools

import jax
import jax.numpy as jnp
from jax import lax
from jax.experimental import pallas as pl
from jax.experimental.pallas import tpu as pltpu
from jax.experimental.pallas import tpu_sc as plsc

NUM_EMB = 1_000_000
DIM = 64
B = 4096
S = 200
SP = 256
NW = 32
CH = 128
L = 16
NG = 2
NO = 2
OST = CH + 1

_mesh = plsc.VectorSubcoreMesh(core_axis_name="c", subcore_axis_name="s")


@functools.partial(
    pl.kernel,
    mesh=_mesh,
    out_type=jax.ShapeDtypeStruct((S, DIM, B), jnp.float32),
    scratch_types=(
        [
            pltpu.VMEM((CH, SP), jnp.int32),
            pltpu.VMEM((S, CH), jnp.int32),
            pltpu.VMEM((NG, CH), jnp.int32),
        ]
        + [pltpu.VMEM((CH, 2 * DIM), jnp.float32) for _ in range(NG)]
        + [pltpu.VMEM((DIM, OST), jnp.float32) for _ in range(NO)]
        + [pltpu.SemaphoreType.DMA for _ in range(NG + NO)]
    ),
    compiler_params=pltpu.CompilerParams(
        use_tc_tiling_on_sc=True, needs_layout_passes=False
    ),
)
def _gather_kernel(idx_hbm, w_hbm, out_hbm, idxT_v, idx_v, q_v, *rest):
    gbuf = rest[:NG]
    obuf = rest[NG:NG + NO]
    gsem = rest[NG + NO:2 * NG + NO]
    ssem = rest[2 * NG + NO:]

    wid = lax.axis_index("s") * 2 + lax.axis_index("c")
    b0 = wid * CH

    pltpu.sync_copy(idx_hbm.at[wid], idxT_v)

    rows = [lax.iota(jnp.int32, L) + (L * gi) for gi in range(CH // L)]
    cvecs = [lax.iota(jnp.int32, L) + (L * kk) for kk in range(DIM // L)]

    def idx_body(j, jvec):
        for gi in range(CH // L):
            v = plsc.load_gather(idxT_v, [rows[gi], jvec])
            idx_v[j, pl.ds(L * gi, L)] = v
        return jvec + 1

    lax.fori_loop(0, S, idx_body, jnp.zeros((L,), jnp.int32), unroll=False)

    def start_gather(k, j):
        for gi in range(CH // L):
            q_v[k, pl.ds(L * gi, L)] = (
                idx_v[j, pl.ds(L * gi, L)] >> 1
            )
        pltpu.async_copy(w_hbm.at[q_v.at[k]], gbuf[k], gsem[k])

    def wait_gather(k):
        pltpu.make_async_copy(
            w_hbm.at[q_v.at[0]], gbuf[k], gsem[k]
        ).wait()

    def start_scatter(k, j):
        pltpu.async_copy(
            obuf[k].at[:, pl.ds(0, CH)],
            out_hbm.at[j, :, pl.ds(b0, CH)],
            ssem[k],
        )

    def wait_scatter(k):
        pltpu.make_async_copy(
            obuf[k].at[:, pl.ds(0, CH)],
            out_hbm.at[0, :, pl.ds(b0, CH)],
            ssem[k],
        ).wait()

    def select_transpose(kg, ko, jvec):
        def body(i, ivec):
            pvec = plsc.load_gather(idx_v, [jvec, ivec])
            odd = (pvec & 1) != 0
            row = gbuf[kg].at[i]
            for kk in range(DIM // L):
                v0 = row[pl.ds(L * kk, L)]
                v1 = row[pl.ds(DIM + L * kk, L)]
                v = jnp.where(odd, v1, v0)
                plsc.store_scatter(obuf[ko], [cvecs[kk], ivec], v)
            return ivec + 1

        lax.fori_loop(0, CH, body, jnp.zeros((L,), jnp.int32),
                      unroll=False)

    for j in range(NG):
        start_gather(j, j)

    def step(j, jvec, kg, ko, swait, gissue):
        if swait:
            wait_scatter(ko)
        wait_gather(kg)
        select_transpose(kg, ko, jvec)
        start_scatter(ko, j)
        if gissue:
            start_gather(kg, j + NG)

    def cvec_of(j):
        return jnp.full((L,), j, dtype=jnp.int32)

    for j in range(2 * NG):
        step(j, cvec_of(j), j % NG, j % NO, j >= NO, True)

    def group_body(g, jvec):
        for k in range(2 * NG):
            j = 2 * NG * g + k
            step(j, jvec, k % NG, k % NO, True, True)
            jvec = jvec + 1
        return jvec

    n_groups = S // (2 * NG) - 2
    jvec = lax.fori_loop(1, n_groups + 1, group_body,
                         cvec_of(2 * NG), unroll=False)

    tail_start = 2 * NG * (n_groups + 1)
    for j in range(tail_start, S):
        step(j, cvec_of(j), j % NG, j % NO, True, j + NG < S)
    for k in range(NO):
        wait_scatter(k)


def kernel(token_ids, W):
    idx3 = token_ids.astype(jnp.int32).reshape(NW, CH, S)
    idx3 = jnp.pad(idx3, ((0, 0), (0, 0), (0, SP - S)))
    w2 = W.reshape(NUM_EMB // 2, 2 * DIM)
    out = _gather_kernel(idx3, w2)
    return jnp.transpose(out, (2, 0, 1))


# speedup vs baseline: 1.7371x; 1.7371x over previous
import functools

import jax
import jax.numpy as jnp
from jax import lax
from jax.experimental import pallas as pl
from jax.experimental.pallas import tpu as pltpu
from jax.experimental.pallas import tpu_sc as plsc

NUM_EMB = 1_000_000
DIM = 64
PDIM = 128
BATCH = 4096 * 200
NW = 32
CH = 128
NBUF = 4
LAG = 2
B_PER_W = BATCH // NW
N_CH = B_PER_W // CH

_mesh = plsc.VectorSubcoreMesh(core_axis_name="c", subcore_axis_name="s")


@functools.partial(
    pl.kernel,
    mesh=_mesh,
    out_type=jax.ShapeDtypeStruct((BATCH, PDIM), jnp.float32),
    scratch_types=(
        [pltpu.VMEM((N_CH, CH), jnp.int32)]
        + [pltpu.VMEM((CH, PDIM), jnp.float32) for _ in range(NBUF)]
        + [pltpu.SemaphoreType.DMA for _ in range(2 * NBUF)]
    ),
)
def _gather_kernel(idx_hbm, w_hbm, out_hbm, idx_v, *rest):
    bufs = list(rest[:NBUF])
    gsem = list(rest[NBUF:2 * NBUF])
    ssem = list(rest[2 * NBUF:])

    wid = lax.axis_index("s") * 2 + lax.axis_index("c")
    base = wid * B_PER_W

    pltpu.sync_copy(idx_hbm.at[wid], idx_v)

    def start_gather(b, j):
        pltpu.async_copy(w_hbm.at[idx_v.at[j]], bufs[b], gsem[b])

    def wait_gather(b):
        pltpu.make_async_copy(
            w_hbm.at[idx_v.at[0]], bufs[b], gsem[b]
        ).wait()

    def start_scatter(b, j):
        pltpu.async_copy(bufs[b], out_hbm.at[pl.ds(base + j * CH, CH)], ssem[b])

    def wait_scatter(b):
        pltpu.make_async_copy(
            bufs[b], out_hbm.at[pl.ds(base, CH)], ssem[b]
        ).wait()


    for j in range(NBUF):
        start_gather(j % NBUF, j)
        if j >= LAG:
            b2 = (j - LAG) % NBUF
            wait_gather(b2)
            start_scatter(b2, j - LAG)

    def group_body(g, carry):
        j0 = (g + 1) * NBUF
        for k in range(NBUF):
            j = j0 + k
            wait_scatter(k)
            start_gather(k, j)
            b2 = (k - LAG) % NBUF
            wait_gather(b2)
            start_scatter(b2, j - LAG)
        return carry

    lax.fori_loop(0, N_CH // NBUF - 1, group_body, 0, unroll=False)

    for j in range(N_CH, N_CH + LAG):
        b2 = (j - LAG) % NBUF
        wait_gather(b2)
        start_scatter(b2, j - LAG)
    for b in range(NBUF):
        wait_scatter(b)


def kernel(token_ids, W):
    idx = token_ids.astype(jnp.int32).reshape(NW, N_CH, CH)
    w_pad = jnp.pad(W, ((0, 0), (0, PDIM - DIM)))
    out = _gather_kernel(idx, w_pad)
    out = out.reshape(token_ids.shape[0], token_ids.shape[1], PDIM)
    return out[:, :, :DIM]
